# Initial kernel scaffold; baseline (speedup 1.0000x reference)
import jax
import jax.numpy as jnp
from jax.experimental import pallas as pl


def kernel(anchor_boxes, pred_reg, cls_scores):
    raise NotImplementedError("write your pallas kernel here")


# speedup vs baseline: 6.5936x; 6.5936x over previous
import dataclasses
import math

import jax
import jax.numpy as jnp
from jax import lax
from jax.experimental import pallas as pl
from jax.experimental.pallas import tpu as pltpu
from jax.experimental.pallas import tpu_sc as plsc

_SCALE_CLAMP = math.log(1000.0 / 16.0)
_CTR_CLAMP = 32.0
_CONF = 0.05
_NMS_T = 0.6
_NCLS = 20
_KEEP = 100
_N = 20000
_T = 16
_CHUNK = 1280
_S = 80
_CP = _S * 16
_LAST = _N - (_T - 1) * _CHUNK
_NF = 16
_L = 16
_BIG = 3.0e38


def _sigmoid(x):
    return 1.0 / (1.0 + jnp.exp(-x))


def _body(anch, reg, cls_s, out, a_v, r_v, c_v, fld, rec_v, all_v, out_v, shared):
    cid = lax.axis_index("c")
    tid = lax.axis_index("s")
    iota_i = lax.iota(jnp.int32, _L)
    iota_f = iota_i.astype(jnp.float32)
    base = tid * _CHUNK

    @pl.when(tid < _T - 1)
    def _():
        pltpu.sync_copy(anch.at[pl.ds(base, _CHUNK)], a_v)
        pltpu.sync_copy(reg.at[pl.ds(base, _CHUNK)], r_v)
        pltpu.sync_copy(cls_s.at[pl.ds(base, _CHUNK)], c_v)

    @pl.when(tid == _T - 1)
    def _():
        pltpu.sync_copy(anch.at[pl.ds(base, _LAST)], a_v.at[pl.ds(0, _LAST)])
        pltpu.sync_copy(reg.at[pl.ds(base, _LAST)], r_v.at[pl.ds(0, _LAST)])
        pltpu.sync_copy(cls_s.at[pl.ds(base, _LAST)], c_v.at[pl.ds(0, _LAST)])

    @pl.loop(0, _S)
    def _stage1(s):
        o = s * _L
        row = o + iota_i

        def g(ref, col):
            return plsc.load_gather(ref, [row, jnp.full((_L,), col, jnp.int32)])

        aw = g(a_v, 2)
        ah = g(a_v, 3)
        ox = jnp.clip(g(r_v, 0) * aw, -_CTR_CLAMP, _CTR_CLAMP)
        oy = jnp.clip(g(r_v, 1) * ah, -_CTR_CLAMP, _CTR_CLAMP)
        cx = g(a_v, 0) + ox
        cy = g(a_v, 1) + oy
        wx = aw * jnp.exp(jnp.minimum(g(r_v, 2), _SCALE_CLAMP))
        wy = ah * jnp.exp(jnp.minimum(g(r_v, 3), _SCALE_CLAMP))
        bx1 = cx - 0.5 * wx
        by1 = cy - 0.5 * wy
        bx2 = cx + 0.5 * wx
        by2 = cy + 0.5 * wy
        bestp = _sigmoid(g(c_v, 0))
        bestc = jnp.zeros((_L,), jnp.float32)
        for kc in range(1, _NCLS):
            p = _sigmoid(g(c_v, kc))
            upd = p > bestp
            bestp = jnp.where(upd, p, bestp)
            bestc = jnp.where(upd, float(kc), bestc)
        score = jnp.where(bestp >= _CONF, bestp, 0.0)
        score = jnp.where(base + row < _N, score, 0.0)
        offs = bestc * 4096.0
        ox1 = bx1 + offs
        oy1 = by1 + offs
        ox2 = bx2 + offs
        oy2 = by2 + offs
        fld[pl.ds(0 * _CP + o, _L)] = score
        fld[pl.ds(1 * _CP + o, _L)] = (base + row).astype(jnp.float32)
        fld[pl.ds(2 * _CP + o, _L)] = ox1
        fld[pl.ds(3 * _CP + o, _L)] = oy1
        fld[pl.ds(4 * _CP + o, _L)] = ox2
        fld[pl.ds(5 * _CP + o, _L)] = oy2
        fld[pl.ds(6 * _CP + o, _L)] = (ox2 - ox1) * (oy2 - oy1)
        fld[pl.ds(7 * _CP + o, _L)] = bx1
        fld[pl.ds(8 * _CP + o, _L)] = by1
        fld[pl.ds(9 * _CP + o, _L)] = bx2
        fld[pl.ds(10 * _CP + o, _L)] = by2

    bm0 = jnp.full((_L,), -1.0, jnp.float32)
    bi0 = jnp.zeros((_L,), jnp.float32)

    def _lane_finish(bm, bi):
        m = jnp.max(bm)
        return m, jnp.min(jnp.where(bm == m, bi, _BIG))

    def _amax(s, carry):
        bm, bi = carry
        o = s * _L
        sc = fld[pl.ds(o, _L)]
        gi = fld[pl.ds(_CP + o, _L)]
        upd = sc > bm
        return jnp.where(upd, sc, bm), jnp.where(upd, gi, bi)

    m0, li0 = _lane_finish(*lax.fori_loop(0, _S, _amax, (bm0, bi0)))

    def _iter(k, carry):
        m, li = carry
        j = li.astype(jnp.int32) - base
        rec_v[...] = plsc.load_gather(fld, [iota_i * _CP + j])
        slot = jnp.bitwise_and(k, 1) * (_T * _L)
        pltpu.sync_copy(rec_v, shared.at[pl.ds(slot + tid * _L, _L)])
        plsc.subcore_barrier()
        pltpu.sync_copy(shared.at[pl.ds(slot, _T * _L)], all_v)
        sc_v = plsc.load_gather(all_v, [iota_i * _L])
        gi_v = plsc.load_gather(all_v, [iota_i * _L + 1])
        gm = jnp.max(sc_v)
        elig = sc_v == gm
        gsel = jnp.min(jnp.where(elig, gi_v, _BIG))
        tf = jnp.min(jnp.where(elig & (gi_v == gsel), iota_f, _BIG))
        tb = tf.astype(jnp.int32) * _L

        def bf(fi):
            return plsc.load_gather(all_v, [jnp.full((_L,), tb + fi, jnp.int32)])

        wx1 = bf(2)
        wy1 = bf(3)
        wx2 = bf(4)
        wy2 = bf(5)
        war = bf(6)

        @pl.when(tid == 0)
        def _():
            perm = jnp.where(iota_i < 4, iota_i + 7, 0)
            rowv = plsc.load_gather(all_v, [tb + perm])
            plsc.store_scatter(out_v, [jnp.full((_L,), k, jnp.int32), iota_i], rowv)

        gsv = jnp.full((_L,), gsel, jnp.float32)

        def _sup(s, carry2):
            bm, bi = carry2
            o = s * _L
            sc = fld[pl.ds(o, _L)]
            gi = fld[pl.ds(_CP + o, _L)]
            x1 = fld[pl.ds(2 * _CP + o, _L)]
            y1 = fld[pl.ds(3 * _CP + o, _L)]
            x2 = fld[pl.ds(4 * _CP + o, _L)]
            y2 = fld[pl.ds(5 * _CP + o, _L)]
            ar = fld[pl.ds(6 * _CP + o, _L)]
            xx1 = jnp.maximum(wx1, x1)
            yy1 = jnp.maximum(wy1, y1)
            xx2 = jnp.minimum(wx2, x2)
            yy2 = jnp.minimum(wy2, y2)
            w = jnp.maximum(1e-28, xx2 - xx1)
            h = jnp.maximum(1e-28, yy2 - yy1)
            inter = w * h
            iou = inter / (war + ar - inter + 1e-14)
            kill = (iou > _NMS_T) | (gi == gsv)
            sc = jnp.where(kill, 0.0, sc)
            fld[pl.ds(o, _L)] = sc
            upd = sc > bm
            return jnp.where(upd, sc, bm), jnp.where(upd, gi, bi)

        return _lane_finish(*lax.fori_loop(0, _S, _sup, (bm0, bi0)))

    lax.fori_loop(0, _KEEP, _iter, (m0, li0))

    @pl.when((cid == 0) & (tid == 0))
    def _():
        pltpu.sync_copy(out_v, out)


@jax.jit
def _nms(anchor_boxes, pred_reg, cls_scores):
    mesh = plsc.VectorSubcoreMesh(
        core_axis_name="c", subcore_axis_name="s", num_cores=2, num_subcores=_T
    )
    cp = pltpu.CompilerParams(use_tc_tiling_on_sc=False)
    if "needs_layout_passes" in pltpu.CompilerParams.__dataclass_fields__:
        cp = dataclasses.replace(cp, needs_layout_passes=False)
    f = pl.kernel(
        _body,
        out_type=jax.ShapeDtypeStruct((_KEEP, _L), jnp.float32),
        mesh=mesh,
        compiler_params=cp,
        scratch_types=[
            pltpu.VMEM((_CP, 4), jnp.float32),
            pltpu.VMEM((_CP, 4), jnp.float32),
            pltpu.VMEM((_CP, _NCLS), jnp.float32),
            pltpu.VMEM((_NF * _CP,), jnp.float32),
            pltpu.VMEM((_L,), jnp.float32),
            pltpu.VMEM((_T * _L,), jnp.float32),
            pltpu.VMEM((_KEEP, _L), jnp.float32),
            pltpu.VMEM_SHARED((2 * _T * _L,), jnp.float32),
        ],
    )
    out16 = f(anchor_boxes, pred_reg, cls_scores)
    return out16[:, :5]


def kernel(anchor_boxes, pred_reg, cls_scores):
    return _nms(anchor_boxes, pred_reg, cls_scores)


# speedup vs baseline: 6.6358x; 1.0064x over previous
import dataclasses
import math

import jax
import jax.numpy as jnp
from jax import lax
from jax.experimental import pallas as pl
from jax.experimental.pallas import tpu as pltpu
from jax.experimental.pallas import tpu_sc as plsc

_SCALE_CLAMP = math.log(1000.0 / 16.0)
_CTR_CLAMP = 32.0
_CONF = 0.05
_NMS_T = 0.6
_NCLS = 20
_KEEP = 100
_N = 20000
_T = 16
_CHUNK = 1280
_S = 80
_CP = _S * 16
_LAST = _N - (_T - 1) * _CHUNK
_NF = 16
_L = 16
_BIG = 3.0e38


def _sigmoid(x):
    return 1.0 / (1.0 + jnp.exp(-x))


def _body(anch, reg, cls_s, out, a_v, r_v, c_v, fld, rec_v, all_v, out_v, shared):
    cid = lax.axis_index("c")
    tid = lax.axis_index("s")
    iota_i = lax.iota(jnp.int32, _L)
    iota_f = iota_i.astype(jnp.float32)
    base = tid * _CHUNK

    @pl.when(tid < _T - 1)
    def _():
        pltpu.sync_copy(anch.at[pl.ds(base, _CHUNK)], a_v)
        pltpu.sync_copy(reg.at[pl.ds(base, _CHUNK)], r_v)
        pltpu.sync_copy(cls_s.at[pl.ds(base, _CHUNK)], c_v)

    @pl.when(tid == _T - 1)
    def _():
        pltpu.sync_copy(anch.at[pl.ds(base, _LAST)], a_v.at[pl.ds(0, _LAST)])
        pltpu.sync_copy(reg.at[pl.ds(base, _LAST)], r_v.at[pl.ds(0, _LAST)])
        pltpu.sync_copy(cls_s.at[pl.ds(base, _LAST)], c_v.at[pl.ds(0, _LAST)])

    @pl.loop(0, _S)
    def _stage1(s):
        o = s * _L
        row = o + iota_i

        def g(ref, col):
            return plsc.load_gather(ref, [row, jnp.full((_L,), col, jnp.int32)])

        aw = g(a_v, 2)
        ah = g(a_v, 3)
        ox = jnp.clip(g(r_v, 0) * aw, -_CTR_CLAMP, _CTR_CLAMP)
        oy = jnp.clip(g(r_v, 1) * ah, -_CTR_CLAMP, _CTR_CLAMP)
        cx = g(a_v, 0) + ox
        cy = g(a_v, 1) + oy
        wx = aw * jnp.exp(jnp.minimum(g(r_v, 2), _SCALE_CLAMP))
        wy = ah * jnp.exp(jnp.minimum(g(r_v, 3), _SCALE_CLAMP))
        bx1 = cx - 0.5 * wx
        by1 = cy - 0.5 * wy
        bx2 = cx + 0.5 * wx
        by2 = cy + 0.5 * wy
        bestp = _sigmoid(g(c_v, 0))
        bestc = jnp.zeros((_L,), jnp.float32)
        for kc in range(1, _NCLS):
            p = _sigmoid(g(c_v, kc))
            upd = p > bestp
            bestp = jnp.where(upd, p, bestp)
            bestc = jnp.where(upd, float(kc), bestc)
        score = jnp.where(bestp >= _CONF, bestp, 0.0)
        score = jnp.where(base + row < _N, score, 0.0)
        offs = bestc * 4096.0
        ox1 = bx1 + offs
        oy1 = by1 + offs
        ox2 = bx2 + offs
        oy2 = by2 + offs
        fld[pl.ds(0 * _CP + o, _L)] = score
        fld[pl.ds(1 * _CP + o, _L)] = (base + row).astype(jnp.float32)
        fld[pl.ds(2 * _CP + o, _L)] = ox1
        fld[pl.ds(3 * _CP + o, _L)] = oy1
        fld[pl.ds(4 * _CP + o, _L)] = ox2
        fld[pl.ds(5 * _CP + o, _L)] = oy2
        fld[pl.ds(6 * _CP + o, _L)] = (ox2 - ox1) * (oy2 - oy1)
        fld[pl.ds(7 * _CP + o, _L)] = bx1
        fld[pl.ds(8 * _CP + o, _L)] = by1
        fld[pl.ds(9 * _CP + o, _L)] = bx2
        fld[pl.ds(10 * _CP + o, _L)] = by2

    bm0 = jnp.full((_L,), -1.0, jnp.float32)
    bi0 = jnp.zeros((_L,), jnp.float32)

    def _lane_finish(bm, bi):
        m = jnp.max(bm)
        return m, jnp.min(jnp.where(bm == m, bi, _BIG))

    def _amax(s, carry):
        bm, bi = carry
        o = s * _L
        sc = fld[pl.ds(o, _L)]
        gi = fld[pl.ds(_CP + o, _L)]
        upd = sc > bm
        return jnp.where(upd, sc, bm), jnp.where(upd, gi, bi)

    m0, li0 = _lane_finish(*lax.fori_loop(0, _S, _amax, (bm0, bi0), unroll=4))

    def _iter(k, carry):
        m, li = carry
        j = li.astype(jnp.int32) - base
        rec_v[...] = plsc.load_gather(fld, [iota_i * _CP + j])
        slot = jnp.bitwise_and(k, 1) * (_T * _L)
        pltpu.sync_copy(rec_v, shared.at[pl.ds(slot + tid * _L, _L)])
        plsc.subcore_barrier()
        pltpu.sync_copy(shared.at[pl.ds(slot, _T * _L)], all_v)
        sc_v = plsc.load_gather(all_v, [iota_i * _L])
        gi_v = plsc.load_gather(all_v, [iota_i * _L + 1])
        gm = jnp.max(sc_v)
        elig = sc_v == gm
        gsel = jnp.min(jnp.where(elig, gi_v, _BIG))
        tf = jnp.min(jnp.where(elig & (gi_v == gsel), iota_f, _BIG))
        tb = tf.astype(jnp.int32) * _L

        def bf(fi):
            return plsc.load_gather(all_v, [jnp.full((_L,), tb + fi, jnp.int32)])

        wx1 = bf(2)
        wy1 = bf(3)
        wx2 = bf(4)
        wy2 = bf(5)
        war = bf(6)

        @pl.when(tid == 0)
        def _():
            perm = jnp.where(iota_i < 4, iota_i + 7, 0)
            rowv = plsc.load_gather(all_v, [tb + perm])
            plsc.store_scatter(out_v, [jnp.full((_L,), k, jnp.int32), iota_i], rowv)

        gsv = jnp.full((_L,), gsel, jnp.float32)

        def _sup(s, carry2):
            bm, bi = carry2
            o = s * _L
            sc = fld[pl.ds(o, _L)]
            gi = fld[pl.ds(_CP + o, _L)]
            x1 = fld[pl.ds(2 * _CP + o, _L)]
            y1 = fld[pl.ds(3 * _CP + o, _L)]
            x2 = fld[pl.ds(4 * _CP + o, _L)]
            y2 = fld[pl.ds(5 * _CP + o, _L)]
            ar = fld[pl.ds(6 * _CP + o, _L)]
            xx1 = jnp.maximum(wx1, x1)
            yy1 = jnp.maximum(wy1, y1)
            xx2 = jnp.minimum(wx2, x2)
            yy2 = jnp.minimum(wy2, y2)
            w = jnp.maximum(1e-28, xx2 - xx1)
            h = jnp.maximum(1e-28, yy2 - yy1)
            inter = w * h
            iou = inter / (war + ar - inter + 1e-14)
            kill = (iou > _NMS_T) | (gi == gsv)
            sc = jnp.where(kill, 0.0, sc)
            fld[pl.ds(o, _L)] = sc
            upd = sc > bm
            return jnp.where(upd, sc, bm), jnp.where(upd, gi, bi)

        return _lane_finish(*lax.fori_loop(0, _S, _sup, (bm0, bi0), unroll=4))

    lax.fori_loop(0, _KEEP, _iter, (m0, li0))

    @pl.when((cid == 0) & (tid == 0))
    def _():
        pltpu.sync_copy(out_v, out)


@jax.jit
def _nms(anchor_boxes, pred_reg, cls_scores):
    mesh = plsc.VectorSubcoreMesh(
        core_axis_name="c", subcore_axis_name="s", num_cores=2, num_subcores=_T
    )
    cp = pltpu.CompilerParams(use_tc_tiling_on_sc=False)
    if "needs_layout_passes" in pltpu.CompilerParams.__dataclass_fields__:
        cp = dataclasses.replace(cp, needs_layout_passes=False)
    f = pl.kernel(
        _body,
        out_type=jax.ShapeDtypeStruct((_KEEP, _L), jnp.float32),
        mesh=mesh,
        compiler_params=cp,
        scratch_types=[
            pltpu.VMEM((_CP, 4), jnp.float32),
            pltpu.VMEM((_CP, 4), jnp.float32),
            pltpu.VMEM((_CP, _NCLS), jnp.float32),
            pltpu.VMEM((_NF * _CP,), jnp.float32),
            pltpu.VMEM((_L,), jnp.float32),
            pltpu.VMEM((_T * _L,), jnp.float32),
            pltpu.VMEM((_KEEP, _L), jnp.float32),
            pltpu.VMEM_SHARED((2 * _T * _L,), jnp.float32),
        ],
    )
    out16 = f(anchor_boxes, pred_reg, cls_scores)
    return out16[:, :5]


def kernel(anchor_boxes, pred_reg, cls_scores):
    return _nms(anchor_boxes, pred_reg, cls_scores)


# speedup vs baseline: 6.6629x; 1.0041x over previous
import dataclasses
import math

import jax
import jax.numpy as jnp
from jax import lax
from jax.experimental import pallas as pl
from jax.experimental.pallas import tpu as pltpu
from jax.experimental.pallas import tpu_sc as plsc

_SCALE_CLAMP = math.log(1000.0 / 16.0)
_CTR_CLAMP = 32.0
_CONF = 0.05
_NMS_T = 0.6
_NCLS = 20
_KEEP = 100
_N = 20000
_T = 16
_CHUNK = 1280
_S = 80
_CP = _S * 16
_LAST = _N - (_T - 1) * _CHUNK
_NF = 16
_L = 16
_BIG = 3.0e38


def _sigmoid(x):
    return 1.0 / (1.0 + jnp.exp(-x))


def _body(anch, reg, cls_s, out, a_v, r_v, c_v, fld, rec_v, all_v, out_v, shared):
    cid = lax.axis_index("c")

    @pl.when(cid == 0)
    def _core0():
        _core_body(anch, reg, cls_s, out, a_v, r_v, c_v, fld, rec_v, all_v,
                   out_v, shared)


def _core_body(anch, reg, cls_s, out, a_v, r_v, c_v, fld, rec_v, all_v, out_v, shared):
    tid = lax.axis_index("s")
    iota_i = lax.iota(jnp.int32, _L)
    iota_f = iota_i.astype(jnp.float32)
    base = tid * _CHUNK

    @pl.when(tid < _T - 1)
    def _():
        pltpu.sync_copy(anch.at[pl.ds(base, _CHUNK)], a_v)
        pltpu.sync_copy(reg.at[pl.ds(base, _CHUNK)], r_v)
        pltpu.sync_copy(cls_s.at[pl.ds(base, _CHUNK)], c_v)

    @pl.when(tid == _T - 1)
    def _():
        pltpu.sync_copy(anch.at[pl.ds(base, _LAST)], a_v.at[pl.ds(0, _LAST)])
        pltpu.sync_copy(reg.at[pl.ds(base, _LAST)], r_v.at[pl.ds(0, _LAST)])
        pltpu.sync_copy(cls_s.at[pl.ds(base, _LAST)], c_v.at[pl.ds(0, _LAST)])

    @pl.loop(0, _S)
    def _stage1(s):
        o = s * _L
        row = o + iota_i

        def g(ref, col):
            return plsc.load_gather(ref, [row, jnp.full((_L,), col, jnp.int32)])

        aw = g(a_v, 2)
        ah = g(a_v, 3)
        ox = jnp.clip(g(r_v, 0) * aw, -_CTR_CLAMP, _CTR_CLAMP)
        oy = jnp.clip(g(r_v, 1) * ah, -_CTR_CLAMP, _CTR_CLAMP)
        cx = g(a_v, 0) + ox
        cy = g(a_v, 1) + oy
        wx = aw * jnp.exp(jnp.minimum(g(r_v, 2), _SCALE_CLAMP))
        wy = ah * jnp.exp(jnp.minimum(g(r_v, 3), _SCALE_CLAMP))
        bx1 = cx - 0.5 * wx
        by1 = cy - 0.5 * wy
        bx2 = cx + 0.5 * wx
        by2 = cy + 0.5 * wy
        bestp = _sigmoid(g(c_v, 0))
        bestc = jnp.zeros((_L,), jnp.float32)
        for kc in range(1, _NCLS):
            p = _sigmoid(g(c_v, kc))
            upd = p > bestp
            bestp = jnp.where(upd, p, bestp)
            bestc = jnp.where(upd, float(kc), bestc)
        score = jnp.where(bestp >= _CONF, bestp, 0.0)
        score = jnp.where(base + row < _N, score, 0.0)
        offs = bestc * 4096.0
        ox1 = bx1 + offs
        oy1 = by1 + offs
        ox2 = bx2 + offs
        oy2 = by2 + offs
        fld[pl.ds(0 * _CP + o, _L)] = score
        fld[pl.ds(1 * _CP + o, _L)] = (base + row).astype(jnp.float32)
        fld[pl.ds(2 * _CP + o, _L)] = ox1
        fld[pl.ds(3 * _CP + o, _L)] = oy1
        fld[pl.ds(4 * _CP + o, _L)] = ox2
        fld[pl.ds(5 * _CP + o, _L)] = oy2
        fld[pl.ds(6 * _CP + o, _L)] = (ox2 - ox1) * (oy2 - oy1)
        fld[pl.ds(7 * _CP + o, _L)] = bx1
        fld[pl.ds(8 * _CP + o, _L)] = by1
        fld[pl.ds(9 * _CP + o, _L)] = bx2
        fld[pl.ds(10 * _CP + o, _L)] = by2

    bm0 = jnp.full((_L,), -1.0, jnp.float32)
    bi0 = jnp.zeros((_L,), jnp.float32)

    def _lane_finish(bm, bi):
        m = jnp.max(bm)
        return m, jnp.min(jnp.where(bm == m, bi, _BIG))

    def _amax(s, carry):
        bm, bi = carry
        o = s * _L
        sc = fld[pl.ds(o, _L)]
        gi = fld[pl.ds(_CP + o, _L)]
        upd = sc > bm
        return jnp.where(upd, sc, bm), jnp.where(upd, gi, bi)

    m0, li0 = _lane_finish(*lax.fori_loop(0, _S, _amax, (bm0, bi0), unroll=4))

    def _iter(k, carry):
        m, li = carry
        j = li.astype(jnp.int32) - base
        rec_v[...] = plsc.load_gather(fld, [iota_i * _CP + j])
        slot = jnp.bitwise_and(k, 1) * (_T * _L)
        pltpu.sync_copy(rec_v, shared.at[pl.ds(slot + tid * _L, _L)])
        plsc.subcore_barrier()
        pltpu.sync_copy(shared.at[pl.ds(slot, _T * _L)], all_v)
        sc_v = plsc.load_gather(all_v, [iota_i * _L])
        gi_v = plsc.load_gather(all_v, [iota_i * _L + 1])
        gm = jnp.max(sc_v)
        elig = sc_v == gm
        gsel = jnp.min(jnp.where(elig, gi_v, _BIG))
        tf = jnp.min(jnp.where(elig & (gi_v == gsel), iota_f, _BIG))
        tb = tf.astype(jnp.int32) * _L

        def bf(fi):
            return plsc.load_gather(all_v, [jnp.full((_L,), tb + fi, jnp.int32)])

        wx1 = bf(2)
        wy1 = bf(3)
        wx2 = bf(4)
        wy2 = bf(5)
        war = bf(6)

        @pl.when(tid == 0)
        def _():
            perm = jnp.where(iota_i < 4, iota_i + 7, 0)
            rowv = plsc.load_gather(all_v, [tb + perm])
            plsc.store_scatter(out_v, [jnp.full((_L,), k, jnp.int32), iota_i], rowv)

        gsv = jnp.full((_L,), gsel, jnp.float32)

        def _sup(s, carry2):
            bm, bi = carry2
            o = s * _L
            sc = fld[pl.ds(o, _L)]
            gi = fld[pl.ds(_CP + o, _L)]
            x1 = fld[pl.ds(2 * _CP + o, _L)]
            y1 = fld[pl.ds(3 * _CP + o, _L)]
            x2 = fld[pl.ds(4 * _CP + o, _L)]
            y2 = fld[pl.ds(5 * _CP + o, _L)]
            ar = fld[pl.ds(6 * _CP + o, _L)]
            xx1 = jnp.maximum(wx1, x1)
            yy1 = jnp.maximum(wy1, y1)
            xx2 = jnp.minimum(wx2, x2)
            yy2 = jnp.minimum(wy2, y2)
            w = jnp.maximum(1e-28, xx2 - xx1)
            h = jnp.maximum(1e-28, yy2 - yy1)
            inter = w * h
            iou = inter / (war + ar - inter + 1e-14)
            kill = (iou > _NMS_T) | (gi == gsv)
            sc = jnp.where(kill, 0.0, sc)
            fld[pl.ds(o, _L)] = sc
            upd = sc > bm
            return jnp.where(upd, sc, bm), jnp.where(upd, gi, bi)

        return _lane_finish(*lax.fori_loop(0, _S, _sup, (bm0, bi0), unroll=4))

    lax.fori_loop(0, _KEEP, _iter, (m0, li0))

    @pl.when(tid == 0)
    def _():
        pltpu.sync_copy(out_v, out)


@jax.jit
def _nms(anchor_boxes, pred_reg, cls_scores):
    mesh = plsc.VectorSubcoreMesh(
        core_axis_name="c", subcore_axis_name="s", num_cores=2, num_subcores=_T
    )
    cp = pltpu.CompilerParams(use_tc_tiling_on_sc=False)
    if "needs_layout_passes" in pltpu.CompilerParams.__dataclass_fields__:
        cp = dataclasses.replace(cp, needs_layout_passes=False)
    f = pl.kernel(
        _body,
        out_type=jax.ShapeDtypeStruct((_KEEP, _L), jnp.float32),
        mesh=mesh,
        compiler_params=cp,
        scratch_types=[
            pltpu.VMEM((_CP, 4), jnp.float32),
            pltpu.VMEM((_CP, 4), jnp.float32),
            pltpu.VMEM((_CP, _NCLS), jnp.float32),
            pltpu.VMEM((_NF * _CP,), jnp.float32),
            pltpu.VMEM((_L,), jnp.float32),
            pltpu.VMEM((_T * _L,), jnp.float32),
            pltpu.VMEM((_KEEP, _L), jnp.float32),
            pltpu.VMEM_SHARED((2 * _T * _L,), jnp.float32),
        ],
    )
    out16 = f(anchor_boxes, pred_reg, cls_scores)
    return out16[:, :5]


def kernel(anchor_boxes, pred_reg, cls_scores):
    return _nms(anchor_boxes, pred_reg, cls_scores)


# speedup vs baseline: 6.7574x; 1.0142x over previous
import dataclasses
import math

import jax
import jax.numpy as jnp
from jax import lax
from jax.experimental import pallas as pl
from jax.experimental.pallas import tpu as pltpu
from jax.experimental.pallas import tpu_sc as plsc

_SCALE_CLAMP = math.log(1000.0 / 16.0)
_CTR_CLAMP = 32.0
_CONF = 0.05
_NMS_T = 0.6
_NCLS = 20
_KEEP = 100
_N = 20000
_T = 16
_CHUNK = 1280
_S = 80
_CP = _S * 16
_LAST = _N - (_T - 1) * _CHUNK
_NF = 16
_L = 16
_BIG = 3.0e38


def _sigmoid(x):
    return 1.0 / (1.0 + jnp.exp(-x))


def _body(anch, reg, cls_s, out, a_v, r_v, c_v, fld, rec_v, all_v, out_v, shared):
    cid = lax.axis_index("c")

    @pl.when(cid == 0)
    def _core0():
        _core_body(anch, reg, cls_s, out, a_v, r_v, c_v, fld, rec_v, all_v,
                   out_v, shared)


def _core_body(anch, reg, cls_s, out, a_v, r_v, c_v, fld, rec_v, all_v, out_v, shared):
    tid = lax.axis_index("s")
    iota_i = lax.iota(jnp.int32, _L)
    iota_f = iota_i.astype(jnp.float32)
    base = tid * _CHUNK

    @pl.when(tid < _T - 1)
    def _():
        pltpu.sync_copy(anch.at[pl.ds(base, _CHUNK)], a_v)
        pltpu.sync_copy(reg.at[pl.ds(base, _CHUNK)], r_v)
        pltpu.sync_copy(cls_s.at[pl.ds(base, _CHUNK)], c_v)

    @pl.when(tid == _T - 1)
    def _():
        pltpu.sync_copy(anch.at[pl.ds(base, _LAST)], a_v.at[pl.ds(0, _LAST)])
        pltpu.sync_copy(reg.at[pl.ds(base, _LAST)], r_v.at[pl.ds(0, _LAST)])
        pltpu.sync_copy(cls_s.at[pl.ds(base, _LAST)], c_v.at[pl.ds(0, _LAST)])

    @pl.loop(0, _S)
    def _stage1(s):
        o = s * _L
        row = o + iota_i

        def g(ref, col):
            return plsc.load_gather(ref, [row, jnp.full((_L,), col, jnp.int32)])

        aw = g(a_v, 2)
        ah = g(a_v, 3)
        ox = jnp.clip(g(r_v, 0) * aw, -_CTR_CLAMP, _CTR_CLAMP)
        oy = jnp.clip(g(r_v, 1) * ah, -_CTR_CLAMP, _CTR_CLAMP)
        cx = g(a_v, 0) + ox
        cy = g(a_v, 1) + oy
        wx = aw * jnp.exp(jnp.minimum(g(r_v, 2), _SCALE_CLAMP))
        wy = ah * jnp.exp(jnp.minimum(g(r_v, 3), _SCALE_CLAMP))
        bx1 = cx - 0.5 * wx
        by1 = cy - 0.5 * wy
        bx2 = cx + 0.5 * wx
        by2 = cy + 0.5 * wy
        bestp = _sigmoid(g(c_v, 0))
        bestc = jnp.zeros((_L,), jnp.float32)
        for kc in range(1, _NCLS):
            p = _sigmoid(g(c_v, kc))
            upd = p > bestp
            bestp = jnp.where(upd, p, bestp)
            bestc = jnp.where(upd, float(kc), bestc)
        score = jnp.where(bestp >= _CONF, bestp, 0.0)
        score = jnp.where(base + row < _N, score, 0.0)
        offs = bestc * 4096.0
        ox1 = bx1 + offs
        oy1 = by1 + offs
        ox2 = bx2 + offs
        oy2 = by2 + offs
        fld[pl.ds(0 * _CP + o, _L)] = score
        fld[pl.ds(1 * _CP + o, _L)] = (base + row).astype(jnp.float32)
        fld[pl.ds(2 * _CP + o, _L)] = ox1
        fld[pl.ds(3 * _CP + o, _L)] = oy1
        fld[pl.ds(4 * _CP + o, _L)] = ox2
        fld[pl.ds(5 * _CP + o, _L)] = oy2
        fld[pl.ds(6 * _CP + o, _L)] = (ox2 - ox1) * (oy2 - oy1)
        fld[pl.ds(7 * _CP + o, _L)] = bx1
        fld[pl.ds(8 * _CP + o, _L)] = by1
        fld[pl.ds(9 * _CP + o, _L)] = bx2
        fld[pl.ds(10 * _CP + o, _L)] = by2

    bm0 = jnp.full((_L,), -1.0, jnp.float32)
    bi0 = jnp.zeros((_L,), jnp.float32)

    def _lane_finish(bm, bi):
        m = jnp.max(bm)
        return m, jnp.min(jnp.where(bm == m, bi, _BIG))

    def _amax(s, carry):
        bm, bi = carry
        o = s * _L
        sc = fld[pl.ds(o, _L)]
        gi = fld[pl.ds(_CP + o, _L)]
        upd = sc > bm
        return jnp.where(upd, sc, bm), jnp.where(upd, gi, bi)

    m0, li0 = _lane_finish(*lax.fori_loop(0, _S, _amax, (bm0, bi0), unroll=4))

    def _iter(k, carry):
        m, li = carry
        j = li.astype(jnp.int32) - base
        rec_v[...] = plsc.load_gather(fld, [iota_i * _CP + j])
        slot = jnp.bitwise_and(k, 1) * (_T * _L)
        pltpu.sync_copy(rec_v, shared.at[pl.ds(slot + tid * _L, _L)])
        plsc.subcore_barrier()
        pltpu.sync_copy(shared.at[pl.ds(slot, _T * _L)], all_v)
        sc_v = plsc.load_gather(all_v, [iota_i * _L])
        gi_v = plsc.load_gather(all_v, [iota_i * _L + 1])
        gm = jnp.max(sc_v)
        elig = sc_v == gm
        gsel = jnp.min(jnp.where(elig, gi_v, _BIG))
        tf = jnp.min(jnp.where(elig & (gi_v == gsel), iota_f, _BIG))
        tb = tf.astype(jnp.int32) * _L

        def bf(fi):
            return plsc.load_gather(all_v, [jnp.full((_L,), tb + fi, jnp.int32)])

        wx1 = bf(2)
        wy1 = bf(3)
        wx2 = bf(4)
        wy2 = bf(5)
        war = bf(6)

        @pl.when(tid == 0)
        def _():
            perm = jnp.where(iota_i < 4, iota_i + 7, 0)
            rowv = plsc.load_gather(all_v, [tb + perm])
            plsc.store_scatter(out_v, [jnp.full((_L,), k, jnp.int32), iota_i], rowv)

        gsv = jnp.full((_L,), gsel, jnp.float32)

        def _sup(s, carry2):
            bm, bi = carry2
            o = s * _L
            sc = fld[pl.ds(o, _L)]
            gi = fld[pl.ds(_CP + o, _L)]
            x1 = fld[pl.ds(2 * _CP + o, _L)]
            y1 = fld[pl.ds(3 * _CP + o, _L)]
            x2 = fld[pl.ds(4 * _CP + o, _L)]
            y2 = fld[pl.ds(5 * _CP + o, _L)]
            ar = fld[pl.ds(6 * _CP + o, _L)]
            xx1 = jnp.maximum(wx1, x1)
            yy1 = jnp.maximum(wy1, y1)
            xx2 = jnp.minimum(wx2, x2)
            yy2 = jnp.minimum(wy2, y2)
            w = jnp.maximum(1e-28, xx2 - xx1)
            h = jnp.maximum(1e-28, yy2 - yy1)
            inter = w * h
            iou = inter / (war + ar - inter + 1e-14)
            kill = (iou > _NMS_T) | (gi == gsv)
            sc = jnp.where(kill, 0.0, sc)
            fld[pl.ds(o, _L)] = sc
            upd = sc > bm
            return jnp.where(upd, sc, bm), jnp.where(upd, gi, bi)

        return _lane_finish(*lax.fori_loop(0, _S, _sup, (bm0, bi0), unroll=4))

    lax.fori_loop(0, _KEEP, _iter, (m0, li0))

    @pl.when(tid == 0)
    def _():
        pltpu.sync_copy(out_v, out)


@jax.jit
def _nms(anchor_boxes, pred_reg, cls_scores):
    mesh = plsc.VectorSubcoreMesh(
        core_axis_name="c", subcore_axis_name="s", num_cores=1, num_subcores=_T
    )
    cp = pltpu.CompilerParams(use_tc_tiling_on_sc=False)
    if "needs_layout_passes" in pltpu.CompilerParams.__dataclass_fields__:
        cp = dataclasses.replace(cp, needs_layout_passes=False)
    f = pl.kernel(
        _body,
        out_type=jax.ShapeDtypeStruct((_KEEP, _L), jnp.float32),
        mesh=mesh,
        compiler_params=cp,
        scratch_types=[
            pltpu.VMEM((_CP, 4), jnp.float32),
            pltpu.VMEM((_CP, 4), jnp.float32),
            pltpu.VMEM((_CP, _NCLS), jnp.float32),
            pltpu.VMEM((_NF * _CP,), jnp.float32),
            pltpu.VMEM((_L,), jnp.float32),
            pltpu.VMEM((_T * _L,), jnp.float32),
            pltpu.VMEM((_KEEP, _L), jnp.float32),
            pltpu.VMEM_SHARED((2 * _T * _L,), jnp.float32),
        ],
    )
    out16 = f(anchor_boxes, pred_reg, cls_scores)
    return out16[:, :5]


def kernel(anchor_boxes, pred_reg, cls_scores):
    return _nms(anchor_boxes, pred_reg, cls_scores)


# speedup vs baseline: 10.1515x; 1.5023x over previous
import dataclasses
import math

import jax
import jax.numpy as jnp
from jax import lax
from jax.experimental import pallas as pl
from jax.experimental.pallas import tpu as pltpu
from jax.experimental.pallas import tpu_sc as plsc

_SCALE_CLAMP = math.log(1000.0 / 16.0)
_CTR_CLAMP = 32.0
_CONF = 0.05
_NMS_T = 0.6
_NCLS = 20
_KEEP = 100
_N = 20000
_T = 16
_CHUNK = 1280
_S = 80
_CP = _S * 16
_LAST = _N - (_T - 1) * _CHUNK
_NF = 16
_L = 16
_BIG = 3.0e38


def _sigmoid(x):
    return 1.0 / (1.0 + jnp.exp(-x))


def _body(feat, out, in_v, fld, rec_v, all_v, out_v, shared):
    cid = lax.axis_index("c")

    @pl.when(cid == 0)
    def _core0():
        _core_body(feat, out, in_v, fld, rec_v, all_v, out_v, shared)


def _core_body(feat, out, in_v, fld, rec_v, all_v, out_v, shared):
    tid = lax.axis_index("s")
    iota_i = lax.iota(jnp.int32, _L)
    iota_f = iota_i.astype(jnp.float32)
    base = tid * _CHUNK

    @pl.when(tid < _T - 1)
    def _():
        pltpu.sync_copy(feat.at[pl.ds(base * 28, _CHUNK * 28)], in_v)

    @pl.when(tid == _T - 1)
    def _():
        pltpu.sync_copy(feat.at[pl.ds(base * 28, _LAST * 28)],
                        in_v.at[pl.ds(0, _LAST * 28)])

    @pl.loop(0, _S)
    def _stage1(s):
        o = s * _L
        row = o + iota_i
        rowm = row * 28

        def g(col):
            return plsc.load_gather(in_v, [rowm + col])

        aw = g(2)
        ah = g(3)
        ox = jnp.clip(g(4) * aw, -_CTR_CLAMP, _CTR_CLAMP)
        oy = jnp.clip(g(5) * ah, -_CTR_CLAMP, _CTR_CLAMP)
        cx = g(0) + ox
        cy = g(1) + oy
        wx = aw * jnp.exp(jnp.minimum(g(6), _SCALE_CLAMP))
        wy = ah * jnp.exp(jnp.minimum(g(7), _SCALE_CLAMP))
        bx1 = cx - 0.5 * wx
        by1 = cy - 0.5 * wy
        bx2 = cx + 0.5 * wx
        by2 = cy + 0.5 * wy
        bestp = _sigmoid(g(8))
        bestc = jnp.zeros((_L,), jnp.float32)
        for kc in range(1, _NCLS):
            p = _sigmoid(g(8 + kc))
            upd = p > bestp
            bestp = jnp.where(upd, p, bestp)
            bestc = jnp.where(upd, float(kc), bestc)
        score = jnp.where(bestp >= _CONF, bestp, 0.0)
        score = jnp.where(base + row < _N, score, 0.0)
        offs = bestc * 4096.0
        ox1 = bx1 + offs
        oy1 = by1 + offs
        ox2 = bx2 + offs
        oy2 = by2 + offs
        fld[pl.ds(0 * _CP + o, _L)] = score
        fld[pl.ds(1 * _CP + o, _L)] = (base + row).astype(jnp.float32)
        fld[pl.ds(2 * _CP + o, _L)] = ox1
        fld[pl.ds(3 * _CP + o, _L)] = oy1
        fld[pl.ds(4 * _CP + o, _L)] = ox2
        fld[pl.ds(5 * _CP + o, _L)] = oy2
        fld[pl.ds(6 * _CP + o, _L)] = (ox2 - ox1) * (oy2 - oy1)
        fld[pl.ds(7 * _CP + o, _L)] = bx1
        fld[pl.ds(8 * _CP + o, _L)] = by1
        fld[pl.ds(9 * _CP + o, _L)] = bx2
        fld[pl.ds(10 * _CP + o, _L)] = by2

    bm0 = jnp.full((_L,), -1.0, jnp.float32)
    bi0 = jnp.zeros((_L,), jnp.float32)

    def _lane_finish(bm, bi):
        m = jnp.max(bm)
        return m, jnp.min(jnp.where(bm == m, bi, _BIG))

    def _amax(s, carry):
        bm, bi = carry
        o = s * _L
        sc = fld[pl.ds(o, _L)]
        gi = fld[pl.ds(_CP + o, _L)]
        upd = sc > bm
        return jnp.where(upd, sc, bm), jnp.where(upd, gi, bi)

    m0, li0 = _lane_finish(*lax.fori_loop(0, _S, _amax, (bm0, bi0), unroll=4))

    def _iter(k, carry):
        m, li = carry
        j = li.astype(jnp.int32) - base
        rec_v[...] = plsc.load_gather(fld, [iota_i * _CP + j])
        slot = jnp.bitwise_and(k, 1) * (_T * _L)
        pltpu.sync_copy(rec_v, shared.at[pl.ds(slot + tid * _L, _L)])
        plsc.subcore_barrier()
        pltpu.sync_copy(shared.at[pl.ds(slot, _T * _L)], all_v)
        sc_v = plsc.load_gather(all_v, [iota_i * _L])
        gi_v = plsc.load_gather(all_v, [iota_i * _L + 1])
        gm = jnp.max(sc_v)
        elig = sc_v == gm
        gsel = jnp.min(jnp.where(elig, gi_v, _BIG))
        tf = jnp.min(jnp.where(elig & (gi_v == gsel), iota_f, _BIG))
        tb = tf.astype(jnp.int32) * _L

        def bf(fi):
            return plsc.load_gather(all_v, [jnp.full((_L,), tb + fi, jnp.int32)])

        wx1 = bf(2)
        wy1 = bf(3)
        wx2 = bf(4)
        wy2 = bf(5)
        war = bf(6)

        @pl.when(tid == 0)
        def _():
            perm = jnp.where(iota_i < 4, iota_i + 7, 0)
            rowv = plsc.load_gather(all_v, [tb + perm])
            plsc.store_scatter(out_v, [jnp.full((_L,), k, jnp.int32), iota_i], rowv)

        gsv = jnp.full((_L,), gsel, jnp.float32)

        def _sup(s, carry2):
            bm, bi = carry2
            o = s * _L
            sc = fld[pl.ds(o, _L)]
            gi = fld[pl.ds(_CP + o, _L)]
            x1 = fld[pl.ds(2 * _CP + o, _L)]
            y1 = fld[pl.ds(3 * _CP + o, _L)]
            x2 = fld[pl.ds(4 * _CP + o, _L)]
            y2 = fld[pl.ds(5 * _CP + o, _L)]
            ar = fld[pl.ds(6 * _CP + o, _L)]
            xx1 = jnp.maximum(wx1, x1)
            yy1 = jnp.maximum(wy1, y1)
            xx2 = jnp.minimum(wx2, x2)
            yy2 = jnp.minimum(wy2, y2)
            w = jnp.maximum(1e-28, xx2 - xx1)
            h = jnp.maximum(1e-28, yy2 - yy1)
            inter = w * h
            iou = inter / (war + ar - inter + 1e-14)
            kill = (iou > _NMS_T) | (gi == gsv)
            sc = jnp.where(kill, 0.0, sc)
            fld[pl.ds(o, _L)] = sc
            upd = sc > bm
            return jnp.where(upd, sc, bm), jnp.where(upd, gi, bi)

        return _lane_finish(*lax.fori_loop(0, _S, _sup, (bm0, bi0), unroll=4))

    lax.fori_loop(0, _KEEP, _iter, (m0, li0))

    @pl.when(tid == 0)
    def _():
        pltpu.sync_copy(out_v, out)


@jax.jit
def _nms(anchor_boxes, pred_reg, cls_scores):
    mesh = plsc.VectorSubcoreMesh(
        core_axis_name="c", subcore_axis_name="s", num_cores=1, num_subcores=_T
    )
    cp = pltpu.CompilerParams(use_tc_tiling_on_sc=False)
    if "needs_layout_passes" in pltpu.CompilerParams.__dataclass_fields__:
        cp = dataclasses.replace(cp, needs_layout_passes=False)
    f = pl.kernel(
        _body,
        out_type=jax.ShapeDtypeStruct((_KEEP, _L), jnp.float32),
        mesh=mesh,
        compiler_params=cp,
        scratch_types=[
            pltpu.VMEM((_CHUNK * 28,), jnp.float32),
            pltpu.VMEM((_NF * _CP,), jnp.float32),
            pltpu.VMEM((_L,), jnp.float32),
            pltpu.VMEM((_T * _L,), jnp.float32),
            pltpu.VMEM((_KEEP, _L), jnp.float32),
            pltpu.VMEM_SHARED((2 * _T * _L,), jnp.float32),
        ],
    )
    feat = jnp.concatenate(
        [anchor_boxes, pred_reg, cls_scores], axis=1).reshape(-1)
    out16 = f(feat)
    return out16[:, :5]


def kernel(anchor_boxes, pred_reg, cls_scores):
    return _nms(anchor_boxes, pred_reg, cls_scores)
